# baseline (speedup 1.0000x reference)
import functools

import jax
import jax.numpy as jnp
from jax import lax
from jax.experimental import pallas as pl
from jax.experimental.pallas import tpu as pltpu
from jax.experimental.pallas import tpu_sc as plsc

NU = 10000
NI = 10000
NN = NU + NI
DD = 128
EE = 160000
BN_EPS = 1e-5
INV_STD = 1.0 / (1.0 + BN_EPS) ** 0.5

NC, NS = 2, 16
NW = NC * NS
EPT_RAW = (2 * EE) // NW
CH = 64
NCH = 160
SCH = 32
NSC = NCH // SCH
NBUF = 4
EPT = NCH * CH
ACC_ROWS = 10240
DUMMY_ROW = 10000
ZROWS = ACC_ROWS // NS
ORT = NU // NS

@functools.cache
def _mesh():
    return plsc.VectorSubcoreMesh(core_axis_name="c", subcore_axis_name="s",
                                  num_cores=NC, num_subcores=NS)


def _sc_deg_body(dst_hbm, ones_hbm, zz_hbm, out_hbm, didx, ones_v, obuf, acc,
                 sem):
    c = lax.axis_index("c")
    s = lax.axis_index("s")
    pltpu.sync_copy(zz_hbm, acc.at[pl.ds(s * ZROWS, ZROWS)])
    pltpu.sync_copy(ones_hbm, ones_v)
    row0 = (c * NS + s) * NCH
    pltpu.sync_copy(dst_hbm.at[pl.ds(row0, NCH)], didx)
    plsc.subcore_barrier()

    @pl.loop(0, NCH)
    def _(j):
        pltpu.sync_copy(ones_v, acc.at[didx.at[j]], add=True)

    plsc.subcore_barrier()
    base = jnp.where(c == 0, NU, 0)

    @pl.when(s < 10)
    def _():
        pltpu.sync_copy(acc.at[pl.ds(s * 1000, 1000)], obuf)
        pltpu.sync_copy(obuf, out_hbm.at[pl.ds(base + s * 1000, 1000)])


@functools.cache
def _sc_deg():
    return pl.kernel(
        _sc_deg_body,
        out_type=jax.ShapeDtypeStruct((NN,), jnp.float32),
        mesh=_mesh(),
        scratch_types=[
            pltpu.VMEM((NCH, CH), jnp.int32),
            pltpu.VMEM((CH,), jnp.float32),
            pltpu.VMEM((1000,), jnp.float32),
            pltpu.VMEM_SHARED((ACC_ROWS,), jnp.float32),
            pltpu.SemaphoreType.DMA,
        ],
    )


def _sc_scatter_body(h2_hbm, src_hbm, dst_hbm, zz_hbm, out_hbm,
                     sidx, didx, buf0, buf1, buf2, buf3,
                     gs0, gs1, gs2, gs3, ss0, ss1, ss2, ss3, acc):
    c = lax.axis_index("c")
    s = lax.axis_index("s")
    bufs = (buf0, buf1, buf2, buf3)
    gsems = (gs0, gs1, gs2, gs3)
    ssems = (ss0, ss1, ss2, ss3)
    pltpu.sync_copy(zz_hbm, acc.at[pl.ds(s * ZROWS, ZROWS)])
    row0 = (c * NS + s) * NCH
    plsc.subcore_barrier()

    def gather(j, b):
        pltpu.async_copy(h2_hbm.at[sidx.at[j]], bufs[b], gsems[b])

    def wait_gather(j, b):
        pltpu.make_async_copy(h2_hbm.at[sidx.at[j]], bufs[b], gsems[b]).wait()

    def wait_scat(b):
        pltpu.make_async_copy(bufs[b], acc.at[didx.at[0]], ssems[b]).wait()

    @pl.loop(0, NSC)
    def _(t):
        row = row0 + t * SCH
        pltpu.sync_copy(src_hbm.at[pl.ds(row, SCH)], sidx)
        pltpu.sync_copy(dst_hbm.at[pl.ds(row, SCH)], didx)
        gather(0, 0)
        gather(1, 1)

        @pl.loop(0, SCH, step=NBUF)
        def _(j):
            for r in range(NBUF):
                jj = j + r
                b = r
                b2 = (r + 2) % NBUF
                wait_gather(jj, b)

                @pl.when(jj + 2 < SCH)
                def _():
                    gather(jj + 2, b2)

    plsc.subcore_barrier()
    base = jnp.where(c == 0, NU, 0)

    @pl.when(s < 10)
    def _():
        pltpu.sync_copy(acc.at[pl.ds(s * 1000, 1000)],
                        out_hbm.at[pl.ds(base + s * 1000, 1000)])


@functools.cache
def _sc_scatter():
    return pl.kernel(
        _sc_scatter_body,
        out_type=jax.ShapeDtypeStruct((NN, DD), jnp.float32),
        mesh=_mesh(),
        scratch_types=[
            pltpu.VMEM((SCH, CH), jnp.int32),
            pltpu.VMEM((SCH, CH), jnp.int32),
            pltpu.VMEM((CH, DD), jnp.bfloat16),
            pltpu.VMEM((CH, DD), jnp.bfloat16),
            pltpu.VMEM((CH, DD), jnp.bfloat16),
            pltpu.VMEM((CH, DD), jnp.bfloat16),
            pltpu.SemaphoreType.DMA,
            pltpu.SemaphoreType.DMA,
            pltpu.SemaphoreType.DMA,
            pltpu.SemaphoreType.DMA,
            pltpu.SemaphoreType.DMA,
            pltpu.SemaphoreType.DMA,
            pltpu.SemaphoreType.DMA,
            pltpu.SemaphoreType.DMA,
            pltpu.VMEM_SHARED((ACC_ROWS, DD), jnp.float32),
        ],
    )


_RB = 2000


def _dis(deg_blk):
    return jnp.where(deg_blk > 0, lax.rsqrt(deg_blk), 0.0)


def _mm_first_body(x_ref, w_ref, deg_ref, o_ref):
    o_ref[...] = (_dis(deg_ref[...]) * jnp.dot(
        x_ref[...], w_ref[...],
        preferred_element_type=jnp.float32)).astype(jnp.bfloat16)


def _mm_mid_body(agg_ref, deg_ref, b_ref, g_ref, be_ref, w_ref, o_ref):
    dis = _dis(deg_ref[...])
    t = (dis * agg_ref[...] + b_ref[...]) * INV_STD
    t = jnp.maximum(g_ref[...] * t + be_ref[...], 0.0)
    o_ref[...] = (dis * jnp.dot(
        t, w_ref[...],
        preferred_element_type=jnp.float32)).astype(jnp.bfloat16)


def _post_body(agg_ref, deg_ref, b_ref, g_ref, be_ref, o_ref):
    dis = _dis(deg_ref[...])
    t = (dis * agg_ref[...] + b_ref[...]) * INV_STD
    o_ref[...] = jnp.maximum(g_ref[...] * t + be_ref[...], 0.0)


_row_spec = pl.BlockSpec((_RB, DD), lambda i: (i, 0))
_deg_spec = pl.BlockSpec((_RB, 1), lambda i: (i, 0))
_w_spec = pl.BlockSpec((DD, DD), lambda i: (0, 0))
_v_spec = pl.BlockSpec((1, DD), lambda i: (0, 0))
_GRID = NN // _RB

_mm_first = pl.pallas_call(
    _mm_first_body, grid=(_GRID,),
    in_specs=[_row_spec, _w_spec, _deg_spec], out_specs=_row_spec,
    out_shape=jax.ShapeDtypeStruct((NN, DD), jnp.bfloat16))

_mm_mid = pl.pallas_call(
    _mm_mid_body, grid=(_GRID,),
    in_specs=[_row_spec, _deg_spec, _v_spec, _v_spec, _v_spec, _w_spec],
    out_specs=_row_spec,
    out_shape=jax.ShapeDtypeStruct((NN, DD), jnp.bfloat16))

_post = pl.pallas_call(
    _post_body, grid=(_GRID,),
    in_specs=[_row_spec, _deg_spec, _v_spec, _v_spec, _v_spec],
    out_specs=_row_spec,
    out_shape=jax.ShapeDtypeStruct((NN, DD), jnp.float32))


def kernel(x_user, x_item, edge_index_ui, edge_index_iu,
           W1, b1, g1, be1, W2, b2, g2, be2, W3, b3, g3, be3):
    x = jnp.concatenate([x_user, x_item], axis=0)

    src_g = jnp.concatenate(
        [edge_index_ui[0], edge_index_iu[0] + NU]).astype(jnp.int32)
    dst_l = jnp.concatenate(
        [edge_index_ui[1], edge_index_iu[1]]).astype(jnp.int32)
    src2d = jnp.pad(src_g.reshape(NW, EPT_RAW),
                    ((0, 0), (0, EPT - EPT_RAW))).reshape(NW * NCH, CH)
    dst2d = jnp.pad(dst_l.reshape(NW, EPT_RAW),
                    ((0, 0), (0, EPT - EPT_RAW)),
                    constant_values=DUMMY_ROW).reshape(NW * NCH, CH)

    ones_v = jnp.ones((CH,), jnp.float32)
    zz1 = jnp.zeros((ZROWS,), jnp.float32)
    zz2 = jnp.zeros((ZROWS, DD), jnp.float32)

    deg = _sc_deg()(dst2d, ones_v, zz1)
    deg2 = deg.reshape(NN, 1)

    sc_scatter = _sc_scatter()
    h2 = _mm_first(x, W1, deg2)
    agg = sc_scatter(h2, src2d, dst2d, zz2)
    h2 = _mm_mid(agg, deg2, b1.reshape(1, DD), g1.reshape(1, DD),
                 be1.reshape(1, DD), W2)
    agg = sc_scatter(h2, src2d, dst2d, zz2)
    h2 = _mm_mid(agg, deg2, b2.reshape(1, DD), g2.reshape(1, DD),
                 be2.reshape(1, DD), W3)
    agg = sc_scatter(h2, src2d, dst2d, zz2)
    out = _post(agg, deg2, b3.reshape(1, DD), g3.reshape(1, DD),
                be3.reshape(1, DD))
    return (out[:NU], out[NU:])


# speedup vs baseline: 1.0432x; 1.0432x over previous
import functools

import jax
import jax.numpy as jnp
from jax import lax
from jax.experimental import pallas as pl
from jax.experimental.pallas import tpu as pltpu
from jax.experimental.pallas import tpu_sc as plsc

NU = 10000
NI = 10000
NN = NU + NI
DD = 128
EE = 160000
BN_EPS = 1e-5
INV_STD = 1.0 / (1.0 + BN_EPS) ** 0.5

NC, NS = 2, 16
NW = NC * NS
EPT_RAW = (2 * EE) // NW
CH = 64
NCH = 160
SCH = 32
NSC = NCH // SCH
NBUF = 4
EPT = NCH * CH
ACC_ROWS = 10240
DUMMY_ROW = 10000
ZROWS = ACC_ROWS // NS
ORT = NU // NS

@functools.cache
def _mesh():
    return plsc.VectorSubcoreMesh(core_axis_name="c", subcore_axis_name="s",
                                  num_cores=NC, num_subcores=NS)


def _sc_deg_body(dst_hbm, ones_hbm, zz_hbm, out_hbm, didx, ones_v, obuf, acc,
                 sem):
    c = lax.axis_index("c")
    s = lax.axis_index("s")
    pltpu.sync_copy(zz_hbm, acc.at[pl.ds(s * ZROWS, ZROWS)])
    pltpu.sync_copy(ones_hbm, ones_v)
    row0 = (c * NS + s) * NCH
    pltpu.sync_copy(dst_hbm.at[pl.ds(row0, NCH)], didx)
    plsc.subcore_barrier()

    @pl.loop(0, NCH)
    def _(j):
        pltpu.sync_copy(ones_v, acc.at[didx.at[j]], add=True)

    plsc.subcore_barrier()
    base = jnp.where(c == 0, NU, 0)

    @pl.when(s < 10)
    def _():
        pltpu.sync_copy(acc.at[pl.ds(s * 1000, 1000)], obuf)
        pltpu.sync_copy(obuf, out_hbm.at[pl.ds(base + s * 1000, 1000)])


@functools.cache
def _sc_deg():
    return pl.kernel(
        _sc_deg_body,
        out_type=jax.ShapeDtypeStruct((NN,), jnp.float32),
        mesh=_mesh(),
        scratch_types=[
            pltpu.VMEM((NCH, CH), jnp.int32),
            pltpu.VMEM((CH,), jnp.float32),
            pltpu.VMEM((1000,), jnp.float32),
            pltpu.VMEM_SHARED((ACC_ROWS,), jnp.float32),
            pltpu.SemaphoreType.DMA,
        ],
    )


def _sc_scatter_body(h2_hbm, src_hbm, dst_hbm, zz_hbm, out_hbm,
                     sidx, didx, buf0, buf1, buf2,
                     gs0, gs1, gs2, ss0, ss1, ss2, acc):
    c = lax.axis_index("c")
    s = lax.axis_index("s")
    bufs = (buf0, buf1, buf2)
    gsems = (gs0, gs1, gs2)
    ssems = (ss0, ss1, ss2)
    pltpu.sync_copy(zz_hbm, acc.at[pl.ds(s * ZROWS, ZROWS)])
    row0 = (c * NS + s) * NCH
    plsc.subcore_barrier()

    def gather(j, b):
        pltpu.async_copy(h2_hbm.at[sidx.at[j]], bufs[b], gsems[b])

    def wait_gather(j, b):
        pltpu.make_async_copy(h2_hbm.at[sidx.at[j]], bufs[b], gsems[b]).wait()

    def wait_scat(b):
        pltpu.make_async_copy(bufs[b], acc.at[didx.at[0]], ssems[b]).wait()

    @pl.loop(0, NSC)
    def _(t):
        row = row0 + t * SCH
        pltpu.sync_copy(src_hbm.at[pl.ds(row, SCH)], sidx)
        pltpu.sync_copy(dst_hbm.at[pl.ds(row, SCH)], didx)
        gather(0, 0)
        gather(1, 1)

        @pl.loop(0, SCH - 2, step=3)
        def _(j):
            for r in range(3):
                b = r
                jj = j + r
                wait_gather(jj, b)
                pltpu.async_copy(bufs[b], acc.at[didx.at[jj]], ssems[b],
                                 add=True)
                b2 = (r + 2) % 3

                @pl.when(jj >= 1)
                def _():
                    wait_scat(b2)

                gather(jj + 2, b2)

        wait_gather(SCH - 2, (SCH - 2) % 3)
        pltpu.sync_copy(bufs[(SCH - 2) % 3], acc.at[didx.at[SCH - 2]],
                        add=True)
        wait_gather(SCH - 1, (SCH - 1) % 3)
        pltpu.sync_copy(bufs[(SCH - 1) % 3], acc.at[didx.at[SCH - 1]],
                        add=True)
        wait_scat((SCH - 3) % 3)

    plsc.subcore_barrier()
    base = jnp.where(c == 0, NU, 0)

    @pl.when(s < 10)
    def _():
        pltpu.sync_copy(acc.at[pl.ds(s * 1000, 1000)],
                        out_hbm.at[pl.ds(base + s * 1000, 1000)])


@functools.cache
def _sc_scatter():
    return pl.kernel(
        _sc_scatter_body,
        out_type=jax.ShapeDtypeStruct((NN, DD), jnp.float32),
        mesh=_mesh(),
        scratch_types=[
            pltpu.VMEM((SCH, CH), jnp.int32),
            pltpu.VMEM((SCH, CH), jnp.int32),
            pltpu.VMEM((CH, DD), jnp.float32),
            pltpu.VMEM((CH, DD), jnp.float32),
            pltpu.VMEM((CH, DD), jnp.float32),
            pltpu.SemaphoreType.DMA,
            pltpu.SemaphoreType.DMA,
            pltpu.SemaphoreType.DMA,
            pltpu.SemaphoreType.DMA,
            pltpu.SemaphoreType.DMA,
            pltpu.SemaphoreType.DMA,
            pltpu.VMEM_SHARED((ACC_ROWS, DD), jnp.float32),
        ],
    )


_RB = 2000


def _dis(deg_blk):
    return jnp.where(deg_blk > 0, lax.rsqrt(deg_blk), 0.0)


def _mm_first_body(x_ref, w_ref, deg_ref, o_ref):
    o_ref[...] = _dis(deg_ref[...]) * jnp.dot(
        x_ref[...], w_ref[...], preferred_element_type=jnp.float32)


def _mm_mid_body(agg_ref, deg_ref, b_ref, g_ref, be_ref, w_ref, o_ref):
    dis = _dis(deg_ref[...])
    t = (dis * agg_ref[...] + b_ref[...]) * INV_STD
    t = jnp.maximum(g_ref[...] * t + be_ref[...], 0.0)
    o_ref[...] = dis * jnp.dot(t, w_ref[...], preferred_element_type=jnp.float32)


def _post_body(agg_ref, deg_ref, b_ref, g_ref, be_ref, o_ref):
    dis = _dis(deg_ref[...])
    t = (dis * agg_ref[...] + b_ref[...]) * INV_STD
    o_ref[...] = jnp.maximum(g_ref[...] * t + be_ref[...], 0.0)


_row_spec = pl.BlockSpec((_RB, DD), lambda i: (i, 0))
_deg_spec = pl.BlockSpec((_RB, 1), lambda i: (i, 0))
_w_spec = pl.BlockSpec((DD, DD), lambda i: (0, 0))
_v_spec = pl.BlockSpec((1, DD), lambda i: (0, 0))
_GRID = NN // _RB

_mm_first = pl.pallas_call(
    _mm_first_body, grid=(_GRID,),
    in_specs=[_row_spec, _w_spec, _deg_spec], out_specs=_row_spec,
    out_shape=jax.ShapeDtypeStruct((NN, DD), jnp.float32))

_mm_mid = pl.pallas_call(
    _mm_mid_body, grid=(_GRID,),
    in_specs=[_row_spec, _deg_spec, _v_spec, _v_spec, _v_spec, _w_spec],
    out_specs=_row_spec,
    out_shape=jax.ShapeDtypeStruct((NN, DD), jnp.float32))

_post = pl.pallas_call(
    _post_body, grid=(_GRID,),
    in_specs=[_row_spec, _deg_spec, _v_spec, _v_spec, _v_spec],
    out_specs=_row_spec,
    out_shape=jax.ShapeDtypeStruct((NN, DD), jnp.float32))


def kernel(x_user, x_item, edge_index_ui, edge_index_iu,
           W1, b1, g1, be1, W2, b2, g2, be2, W3, b3, g3, be3):
    x = jnp.concatenate([x_user, x_item], axis=0)

    src_g = jnp.concatenate(
        [edge_index_ui[0], edge_index_iu[0] + NU]).astype(jnp.int32)
    dst_l = jnp.concatenate(
        [edge_index_ui[1], edge_index_iu[1]]).astype(jnp.int32)
    src2d = jnp.pad(src_g.reshape(NW, EPT_RAW),
                    ((0, 0), (0, EPT - EPT_RAW))).reshape(NW * NCH, CH)
    dst2d = jnp.pad(dst_l.reshape(NW, EPT_RAW),
                    ((0, 0), (0, EPT - EPT_RAW)),
                    constant_values=DUMMY_ROW).reshape(NW * NCH, CH)

    ones_v = jnp.ones((CH,), jnp.float32)
    zz1 = jnp.zeros((ZROWS,), jnp.float32)
    zz2 = jnp.zeros((ZROWS, DD), jnp.float32)

    deg = _sc_deg()(dst2d, ones_v, zz1)
    deg2 = deg.reshape(NN, 1)

    sc_scatter = _sc_scatter()
    h2 = _mm_first(x, W1, deg2)
    agg = sc_scatter(h2, src2d, dst2d, zz2)
    h2 = _mm_mid(agg, deg2, b1.reshape(1, DD), g1.reshape(1, DD),
                 be1.reshape(1, DD), W2)
    agg = sc_scatter(h2, src2d, dst2d, zz2)
    h2 = _mm_mid(agg, deg2, b2.reshape(1, DD), g2.reshape(1, DD),
                 be2.reshape(1, DD), W3)
    agg = sc_scatter(h2, src2d, dst2d, zz2)
    out = _post(agg, deg2, b3.reshape(1, DD), g3.reshape(1, DD),
                be3.reshape(1, DD))
    return (out[:NU], out[NU:])


# speedup vs baseline: 2.1748x; 2.0848x over previous
import functools

import jax
import jax.numpy as jnp
from jax import lax
from jax.experimental import pallas as pl
from jax.experimental.pallas import tpu as pltpu
from jax.experimental.pallas import tpu_sc as plsc

NU = 10000
NI = 10000
NN = NU + NI
DD = 128
EE = 160000
BN_EPS = 1e-5
INV_STD = 1.0 / (1.0 + BN_EPS) ** 0.5

NC, NS = 2, 16
NW = NC * NS
EPT_RAW = (2 * EE) // NW
CH = 64
NCH = 160
SCH = 32
NSC = NCH // SCH
SCH_LAST = 29
NBUF = 4
EPT = NCH * CH
ACC_ROWS = 10240
DUMMY_ROW = 10000
ZROWS = ACC_ROWS // NS
ORT = NU // NS

@functools.cache
def _mesh():
    return plsc.VectorSubcoreMesh(core_axis_name="c", subcore_axis_name="s",
                                  num_cores=NC, num_subcores=NS)


def _sc_deg_body(dst_hbm, ones_hbm, zz_hbm, out_hbm, didx, ones_v, obuf, acc,
                 sem):
    c = lax.axis_index("c")
    s = lax.axis_index("s")
    pltpu.sync_copy(zz_hbm, acc.at[pl.ds(s * ZROWS, ZROWS)])
    pltpu.sync_copy(ones_hbm, ones_v)
    row0 = (c * NS + s) * NCH
    pltpu.sync_copy(dst_hbm.at[pl.ds(row0, NCH)], didx)
    plsc.subcore_barrier()

    @pl.loop(0, NCH)
    def _(j):
        pltpu.sync_copy(ones_v, acc.at[didx.at[j]], add=True)

    plsc.subcore_barrier()
    base = jnp.where(c == 0, NU, 0)

    @pl.when(s < 10)
    def _():
        pltpu.sync_copy(acc.at[pl.ds(s * 1000, 1000)], obuf)
        pltpu.sync_copy(obuf, out_hbm.at[pl.ds(base + s * 1000, 1000)])


@functools.cache
def _sc_deg():
    return pl.kernel(
        _sc_deg_body,
        out_type=jax.ShapeDtypeStruct((NN,), jnp.float32),
        mesh=_mesh(),
        scratch_types=[
            pltpu.VMEM((NCH, CH), jnp.int32),
            pltpu.VMEM((CH,), jnp.float32),
            pltpu.VMEM((1000,), jnp.float32),
            pltpu.VMEM_SHARED((ACC_ROWS,), jnp.float32),
            pltpu.SemaphoreType.DMA,
        ],
    )


def _sc_scatter_body(h2_hbm, src_hbm, dst_hbm, zz_hbm, out_hbm,
                     sidx, didx, buf0, buf1, buf2,
                     gs0, gs1, gs2, ss0, ss1, ss2, acc):
    c = lax.axis_index("c")
    s = lax.axis_index("s")
    bufs = (buf0, buf1, buf2)
    gsems = (gs0, gs1, gs2)
    ssems = (ss0, ss1, ss2)
    row0 = (c * NS + s) * NCH

    def gather(j, b):
        pltpu.async_copy(h2_hbm.at[sidx.at[j]], bufs[b], gsems[b])

    def wait_gather(j, b):
        pltpu.make_async_copy(h2_hbm.at[sidx.at[j]], bufs[b], gsems[b]).wait()

    def wait_scat(b):
        pltpu.make_async_copy(bufs[b], acc.at[didx.at[0]], ssems[b]).wait()

    def load_idx(row):
        pltpu.sync_copy(src_hbm.at[pl.ds(row, SCH)], sidx)
        pltpu.sync_copy(dst_hbm.at[pl.ds(row, SCH)], didx)

    def run_superchunk(nch):
        @pl.loop(0, nch - 2, step=3)
        def _(j):
            for r in range(3):
                b = r
                jj = j + r
                wait_gather(jj, b)
                pltpu.async_copy(bufs[b], acc.at[didx.at[jj]], ssems[b],
                                 add=True)
                b2 = (r + 2) % 3

                @pl.when(jj >= 1)
                def _():
                    wait_scat(b2)

                gather(jj + 2, b2)

        wait_gather(nch - 2, (nch - 2) % 3)
        pltpu.sync_copy(bufs[(nch - 2) % 3], acc.at[didx.at[nch - 2]],
                        add=True)
        wait_gather(nch - 1, (nch - 1) % 3)
        pltpu.sync_copy(bufs[(nch - 1) % 3], acc.at[didx.at[nch - 1]],
                        add=True)
        wait_scat((nch - 3) % 3)

    load_idx(row0)
    gather(0, 0)
    gather(1, 1)
    pltpu.sync_copy(zz_hbm, acc.at[pl.ds(s * ZROWS, ZROWS)])
    plsc.subcore_barrier()
    run_superchunk(SCH)

    @pl.loop(1, NSC - 1)
    def _(t):
        load_idx(row0 + t * SCH)
        gather(0, 0)
        gather(1, 1)
        run_superchunk(SCH)

    load_idx(row0 + (NSC - 1) * SCH)
    gather(0, 0)
    gather(1, 1)
    run_superchunk(SCH_LAST)

    plsc.subcore_barrier()
    base = jnp.where(c == 0, NU, 0)

    @pl.when(s < 10)
    def _():
        pltpu.sync_copy(acc.at[pl.ds(s * 1000, 1000)],
                        out_hbm.at[pl.ds(base + s * 1000, 1000)])


@functools.cache
def _sc_scatter():
    return pl.kernel(
        _sc_scatter_body,
        out_type=jax.ShapeDtypeStruct((NN, DD), jnp.float32),
        mesh=_mesh(),
        scratch_types=[
            pltpu.VMEM((SCH, CH), jnp.int32),
            pltpu.VMEM((SCH, CH), jnp.int32),
            pltpu.VMEM((CH, DD), jnp.float32),
            pltpu.VMEM((CH, DD), jnp.float32),
            pltpu.VMEM((CH, DD), jnp.float32),
            pltpu.SemaphoreType.DMA,
            pltpu.SemaphoreType.DMA,
            pltpu.SemaphoreType.DMA,
            pltpu.SemaphoreType.DMA,
            pltpu.SemaphoreType.DMA,
            pltpu.SemaphoreType.DMA,
            pltpu.VMEM_SHARED((ACC_ROWS, DD), jnp.float32),
        ],
    )


_RB = 2000


def _dis(deg_blk):
    return jnp.where(deg_blk > 0, lax.rsqrt(deg_blk), 0.0)


def _mm_first_body(x_ref, w_ref, deg_ref, o_ref):
    o_ref[...] = _dis(deg_ref[...]) * jnp.dot(
        x_ref[...], w_ref[...], preferred_element_type=jnp.float32)


def _mm_mid_body(agg_ref, deg_ref, b_ref, g_ref, be_ref, w_ref, o_ref):
    dis = _dis(deg_ref[...])
    t = (dis * agg_ref[...] + b_ref[...]) * INV_STD
    t = jnp.maximum(g_ref[...] * t + be_ref[...], 0.0)
    o_ref[...] = dis * jnp.dot(t, w_ref[...], preferred_element_type=jnp.float32)


def _post_body(agg_ref, deg_ref, b_ref, g_ref, be_ref, o_ref):
    dis = _dis(deg_ref[...])
    t = (dis * agg_ref[...] + b_ref[...]) * INV_STD
    o_ref[...] = jnp.maximum(g_ref[...] * t + be_ref[...], 0.0)


_row_spec = pl.BlockSpec((_RB, DD), lambda i: (i, 0))
_deg_spec = pl.BlockSpec((_RB, 1), lambda i: (i, 0))
_w_spec = pl.BlockSpec((DD, DD), lambda i: (0, 0))
_v_spec = pl.BlockSpec((1, DD), lambda i: (0, 0))
_GRID = NN // _RB

_mm_first = pl.pallas_call(
    _mm_first_body, grid=(_GRID,),
    in_specs=[_row_spec, _w_spec, _deg_spec], out_specs=_row_spec,
    out_shape=jax.ShapeDtypeStruct((NN, DD), jnp.float32))

_mm_mid = pl.pallas_call(
    _mm_mid_body, grid=(_GRID,),
    in_specs=[_row_spec, _deg_spec, _v_spec, _v_spec, _v_spec, _w_spec],
    out_specs=_row_spec,
    out_shape=jax.ShapeDtypeStruct((NN, DD), jnp.float32))

_post = pl.pallas_call(
    _post_body, grid=(_GRID,),
    in_specs=[_row_spec, _deg_spec, _v_spec, _v_spec, _v_spec],
    out_specs=_row_spec,
    out_shape=jax.ShapeDtypeStruct((NN, DD), jnp.float32))


def kernel(x_user, x_item, edge_index_ui, edge_index_iu,
           W1, b1, g1, be1, W2, b2, g2, be2, W3, b3, g3, be3):
    x = jnp.concatenate([x_user, x_item], axis=0)

    src_g = jnp.concatenate(
        [edge_index_ui[0], edge_index_iu[0] + NU]).astype(jnp.int32)
    dst_l = jnp.concatenate(
        [edge_index_ui[1], edge_index_iu[1]]).astype(jnp.int32)
    src2d = jnp.pad(src_g.reshape(NW, EPT_RAW),
                    ((0, 0), (0, EPT - EPT_RAW))).reshape(NW * NCH, CH)
    dst2d = jnp.pad(dst_l.reshape(NW, EPT_RAW),
                    ((0, 0), (0, EPT - EPT_RAW)),
                    constant_values=DUMMY_ROW).reshape(NW * NCH, CH)

    ones_v = jnp.ones((CH,), jnp.float32)
    zz1 = jnp.zeros((ZROWS,), jnp.float32)
    zz2 = jnp.zeros((ZROWS, DD), jnp.float32)

    deg = _sc_deg()(dst2d, ones_v, zz1)
    deg2 = deg.reshape(NN, 1)

    sc_scatter = _sc_scatter()
    h2 = _mm_first(x, W1, deg2)
    agg = sc_scatter(h2, src2d, dst2d, zz2)
    h2 = _mm_mid(agg, deg2, b1.reshape(1, DD), g1.reshape(1, DD),
                 be1.reshape(1, DD), W2)
    agg = sc_scatter(h2, src2d, dst2d, zz2)
    h2 = _mm_mid(agg, deg2, b2.reshape(1, DD), g2.reshape(1, DD),
                 be2.reshape(1, DD), W3)
    agg = sc_scatter(h2, src2d, dst2d, zz2)
    out = _post(agg, deg2, b3.reshape(1, DD), g3.reshape(1, DD),
                be3.reshape(1, DD))
    return (out[:NU], out[NU:])
